# Initial kernel scaffold; baseline (speedup 1.0000x reference)
import jax
import jax.numpy as jnp
from jax.experimental import pallas as pl


def kernel(img):
    raise NotImplementedError("write your pallas kernel here")


# speedup vs baseline: 14.7222x; 14.7222x over previous
import functools

import jax
import jax.numpy as jnp
from jax import lax
from jax.experimental import pallas as pl
from jax.experimental.pallas import tpu as pltpu
from jax.experimental.pallas import tpu_sc as plsc

B, C, H, W = 16, 3, 512, 512
N = B * H * W
NB = 32768
SHIFT = 15
NSUB = 32
PER_SUB = N // NSUB
CHUNK = 8192
NCHUNK = PER_SUB // CHUNK
L = 16
R_HIGH = 10486
R_LOW = 10487

_mesh = plsc.VectorSubcoreMesh(
    core_axis_name="c", subcore_axis_name="s", num_cores=2, num_subcores=16
)


def _gray_body(img_ref, out_ref):
  r = img_ref[0, 0]
  g = img_ref[0, 1]
  b = img_ref[0, 2]
  out_ref[0, 0] = 0.2989 * r + 0.587 * g + 0.114 * b


def _grayscale(img):
  return pl.pallas_call(
      _gray_body,
      out_shape=jax.ShapeDtypeStruct((B, 1, H, W), jnp.float32),
      grid=(B,),
      in_specs=[pl.BlockSpec((1, C, H, W), lambda i: (i, 0, 0, 0))],
      out_specs=pl.BlockSpec((1, 1, H, W), lambda i: (i, 0, 0, 0)),
  )(img)


def _zero_hist(hist):
  def body(i, carry):
    hist[pl.ds(i * L, L)] = jnp.zeros((L,), jnp.int32)
    return carry
  lax.fori_loop(0, NB // L, body, None)


@functools.partial(
    pl.kernel,
    mesh=_mesh,
    out_type=jax.ShapeDtypeStruct((NSUB, NB), jnp.int32),
    scratch_types=[
        pltpu.VMEM((CHUNK,), jnp.float32),
        pltpu.VMEM((NB,), jnp.int32),
    ],
    compiler_params=pltpu.CompilerParams(needs_layout_passes=False),
)
def _sc_hist1(gray_hbm, out_hbm, buf, hist):
  wid = lax.axis_index("s") * 2 + lax.axis_index("c")
  _zero_hist(hist)
  base = wid * PER_SUB

  def chunk_body(c, carry):
    pltpu.sync_copy(gray_hbm.at[pl.ds(base + c * CHUNK, CHUNK)], buf)

    def vec_body(v, carry2):
      x = buf[pl.ds(v * L, L)]
      bits = plsc.bitcast(x, jnp.int32)
      idx = lax.shift_right_logical(bits, SHIFT)
      cnt, last = plsc.scan_count(idx)
      plsc.addupdate_scatter(hist, [idx], cnt, mask=last)
      return carry2

    lax.fori_loop(0, CHUNK // L, vec_body, None)
    return carry

  lax.fori_loop(0, NCHUNK, chunk_body, None)
  pltpu.sync_copy(hist, out_hbm.at[wid])


@functools.partial(
    pl.kernel,
    mesh=_mesh,
    out_type=(
        jax.ShapeDtypeStruct((NSUB, NB), jnp.int32),
        jax.ShapeDtypeStruct((NSUB, NB), jnp.int32),
    ),
    scratch_types=[
        pltpu.VMEM((CHUNK,), jnp.float32),
        pltpu.VMEM((L,), jnp.int32),
        pltpu.VMEM((L,), jnp.int32),
        pltpu.VMEM((NB,), jnp.int32),
        pltpu.VMEM((NB,), jnp.int32),
    ],
    compiler_params=pltpu.CompilerParams(needs_layout_passes=False),
)
def _sc_hist2(gray_hbm, lo1_hbm, lo2_hbm, outa_hbm, outb_hbm,
              buf, lo1v, lo2v, ha, hb):
  wid = lax.axis_index("s") * 2 + lax.axis_index("c")
  pltpu.sync_copy(lo1_hbm, lo1v)
  pltpu.sync_copy(lo2_hbm, lo2v)
  _zero_hist(ha)
  _zero_hist(hb)
  lo1 = lo1v[...]
  lo2 = lo2v[...]
  base = wid * PER_SUB

  def chunk_body(c, carry):
    pltpu.sync_copy(gray_hbm.at[pl.ds(base + c * CHUNK, CHUNK)], buf)

    def vec_body(v, carry2):
      x = buf[pl.ds(v * L, L)]
      bits = plsc.bitcast(x, jnp.int32)
      d1 = bits - lo1
      m1 = (d1 >= 0) & (d1 < NB)
      cnt1, last1 = plsc.scan_count(d1, mask=m1)
      plsc.addupdate_scatter(ha, [d1], cnt1, mask=last1)
      d2 = bits - lo2
      m2 = (d2 >= 0) & (d2 < NB)
      cnt2, last2 = plsc.scan_count(d2, mask=m2)
      plsc.addupdate_scatter(hb, [d2], cnt2, mask=last2)
      return carry2

    lax.fori_loop(0, CHUNK // L, vec_body, None)
    return carry

  lax.fori_loop(0, NCHUNK, chunk_body, None)
  pltpu.sync_copy(ha, outa_hbm.at[wid])
  pltpu.sync_copy(hb, outb_hbm.at[wid])


def _cmp_body(t_ref, gray_ref, out_ref):
  out_ref[0, 0] = (gray_ref[0, 0] >= t_ref[0, 0]).astype(jnp.int32)


def _compare(gray, t):
  return pl.pallas_call(
      _cmp_body,
      out_shape=jax.ShapeDtypeStruct((B, 1, H, W), jnp.int32),
      grid=(B,),
      in_specs=[
          pl.BlockSpec(memory_space=pltpu.SMEM),
          pl.BlockSpec((1, 1, H, W), lambda i: (i, 0, 0, 0)),
      ],
      out_specs=pl.BlockSpec((1, 1, H, W), lambda i: (i, 0, 0, 0)),
  )(t.reshape(1, 1), gray)


def _suffix_count(hist):
  return jnp.cumsum(hist[::-1])[::-1]


def kernel(img):
  gray = _grayscale(img)
  flat = gray.reshape(N)

  hist1 = _sc_hist1(flat)
  h1 = jnp.sum(hist1, axis=0)
  s1 = _suffix_count(h1)

  def window(rank):
    b = jnp.sum((s1 >= rank).astype(jnp.int32)) - 1
    above = s1[b] - h1[b]
    return b << SHIFT, rank - above

  lo_hi, r_hi = window(R_HIGH)
  lo_lo, r_lo = window(R_LOW)

  hist2a, hist2b = _sc_hist2(
      flat,
      jnp.full((L,), lo_hi, dtype=jnp.int32),
      jnp.full((L,), lo_lo, dtype=jnp.int32),
  )

  def resolve(hist2, lo, rank):
    s2 = _suffix_count(jnp.sum(hist2, axis=0))
    p = jnp.sum((s2 >= rank).astype(jnp.int32)) - 1
    return lax.bitcast_convert_type((lo + p).astype(jnp.int32), jnp.float32)

  v_high = resolve(hist2a, lo_hi, r_hi)
  v_low = resolve(hist2b, lo_lo, r_lo)
  t = v_low * 0.75 + v_high * 0.25

  return _compare(gray, t)


# speedup vs baseline: 17.5187x; 1.1900x over previous
import functools

import jax
import jax.numpy as jnp
from jax import lax
from jax.experimental import pallas as pl
from jax.experimental.pallas import tpu as pltpu
from jax.experimental.pallas import tpu_sc as plsc

B, C, H, W = 16, 3, 512, 512
N = B * H * W
NB = 32768
SHIFT = 15
NSUB = 32
PER_SUB = N // NSUB
CHUNK = 8192
NCHUNK = PER_SUB // CHUNK
L = 16
UNROLL = 8
R_HIGH = 10486
R_LOW = 10487

_mesh = plsc.VectorSubcoreMesh(
    core_axis_name="c", subcore_axis_name="s", num_cores=2, num_subcores=16
)
_sc_params = pltpu.CompilerParams(needs_layout_passes=False)


def _gray_body(img_ref, out_ref):
  r = img_ref[0, 0]
  g = img_ref[0, 1]
  b = img_ref[0, 2]
  out_ref[0, 0] = 0.2989 * r + 0.587 * g + 0.114 * b


def _grayscale(img):
  return pl.pallas_call(
      _gray_body,
      out_shape=jax.ShapeDtypeStruct((B, 1, H, W), jnp.float32),
      grid=(B,),
      in_specs=[pl.BlockSpec((1, C, H, W), lambda i: (i, 0, 0, 0))],
      out_specs=pl.BlockSpec((1, 1, H, W), lambda i: (i, 0, 0, 0)),
  )(img)


def _zero_hist(hist, nbins):
  zeros = jnp.zeros((L,), jnp.int32)

  def body(i, carry):
    base = i * L * UNROLL
    for u in range(UNROLL):
      hist[pl.ds(base + u * L, L)] = zeros
    return carry

  lax.fori_loop(0, nbins // (L * UNROLL), body, None)


def _stream_chunks(gray_hbm, base, buf0, buf1, sem0, sem1, process, carry):
  pltpu.async_copy(gray_hbm.at[pl.ds(base, CHUNK)], buf0, sem0)
  pltpu.async_copy(gray_hbm.at[pl.ds(base + CHUNK, CHUNK)], buf1, sem1)

  def wait(buf, sem):
    pltpu.make_async_copy(gray_hbm.at[pl.ds(0, CHUNK)], buf, sem).wait()

  def body(i, c2):
    c = 2 * i
    wait(buf0, sem0)
    c2 = process(buf0, c2)
    pltpu.async_copy(
        gray_hbm.at[pl.ds(base + (c + 2) * CHUNK, CHUNK)], buf0, sem0)
    wait(buf1, sem1)
    c2 = process(buf1, c2)
    pltpu.async_copy(
        gray_hbm.at[pl.ds(base + (c + 3) * CHUNK, CHUNK)], buf1, sem1)
    return c2

  carry = lax.fori_loop(0, NCHUNK // 2 - 1, body, carry)
  wait(buf0, sem0)
  carry = process(buf0, carry)
  wait(buf1, sem1)
  carry = process(buf1, carry)
  return carry


@functools.partial(
    pl.kernel,
    mesh=_mesh,
    out_type=jax.ShapeDtypeStruct((NSUB, NB), jnp.int32),
    scratch_types=[
        pltpu.VMEM((CHUNK,), jnp.float32),
        pltpu.VMEM((CHUNK,), jnp.float32),
        pltpu.VMEM((NB,), jnp.int32),
        pltpu.SemaphoreType.DMA,
        pltpu.SemaphoreType.DMA,
    ],
    compiler_params=_sc_params,
)
def _sc_hist1(gray_hbm, out_hbm, buf0, buf1, hist, sem0, sem1):
  wid = lax.axis_index("s") * 2 + lax.axis_index("c")
  _zero_hist(hist, NB)

  def process(buf, carry):
    def vec_body(v, c2):
      vbase = v * (L * UNROLL)
      for u in range(UNROLL):
        x = buf[pl.ds(vbase + u * L, L)]
        bits = plsc.bitcast(x, jnp.int32)
        idx = lax.shift_right_logical(bits, SHIFT)
        cnt, last = plsc.scan_count(idx)
        plsc.addupdate_scatter(hist, [idx], cnt, mask=last)
      return c2

    return lax.fori_loop(0, CHUNK // (L * UNROLL), vec_body, carry)

  _stream_chunks(gray_hbm, wid * PER_SUB, buf0, buf1, sem0, sem1, process, 0)
  pltpu.sync_copy(hist, out_hbm.at[wid])


@functools.partial(
    pl.kernel,
    mesh=_mesh,
    out_type=(
        jax.ShapeDtypeStruct((NSUB, NB), jnp.int32),
        jax.ShapeDtypeStruct((NSUB, L), jnp.float32),
    ),
    scratch_types=[
        pltpu.VMEM((CHUNK,), jnp.float32),
        pltpu.VMEM((CHUNK,), jnp.float32),
        pltpu.VMEM((L,), jnp.int32),
        pltpu.VMEM((NB,), jnp.int32),
        pltpu.VMEM((L,), jnp.float32),
        pltpu.SemaphoreType.DMA,
        pltpu.SemaphoreType.DMA,
    ],
    compiler_params=_sc_params,
)
def _sc_hist2(gray_hbm, lo_hbm, out_hbm, max_hbm,
              buf0, buf1, lov, hist, maxv, sem0, sem1):
  wid = lax.axis_index("s") * 2 + lax.axis_index("c")
  pltpu.sync_copy(lo_hbm, lov)
  _zero_hist(hist, NB)
  lo = lov[...]
  neg_inf = jnp.full((L,), -jnp.inf, dtype=jnp.float32)

  def process(buf, acc):
    def vec_body(v, acc2):
      vbase = v * (L * UNROLL)
      for u in range(UNROLL):
        x = buf[pl.ds(vbase + u * L, L)]
        bits = plsc.bitcast(x, jnp.int32)
        d = bits - lo
        m = (d >= 0) & (d < NB)
        cnt, last = plsc.scan_count(d, mask=m)
        plsc.addupdate_scatter(hist, [d], cnt, mask=last)
        acc2 = jnp.maximum(acc2, jnp.where(bits < lo, x, neg_inf))
      return acc2

    return lax.fori_loop(0, CHUNK // (L * UNROLL), vec_body, acc)

  acc = _stream_chunks(
      gray_hbm, wid * PER_SUB, buf0, buf1, sem0, sem1, process, neg_inf)
  maxv[...] = acc
  pltpu.sync_copy(hist, out_hbm.at[wid])
  pltpu.sync_copy(maxv, max_hbm.at[wid])


def _cmp_body(t_ref, gray_ref, out_ref):
  out_ref[0, 0] = (gray_ref[0, 0] >= t_ref[0, 0]).astype(jnp.int32)


def _compare(gray, t):
  return pl.pallas_call(
      _cmp_body,
      out_shape=jax.ShapeDtypeStruct((B, 1, H, W), jnp.int32),
      grid=(B,),
      in_specs=[
          pl.BlockSpec(memory_space=pltpu.SMEM),
          pl.BlockSpec((1, 1, H, W), lambda i: (i, 0, 0, 0)),
      ],
      out_specs=pl.BlockSpec((1, 1, H, W), lambda i: (i, 0, 0, 0)),
  )(t.reshape(1, 1), gray)


def _suffix_count(hist):
  return jnp.cumsum(hist[::-1])[::-1]


def kernel(img):
  gray = _grayscale(img)
  flat = gray.reshape(N)

  hist1 = _sc_hist1(flat)
  h1 = jnp.sum(hist1, axis=0)
  s1 = _suffix_count(h1)

  b_hi = jnp.sum((s1 >= R_HIGH).astype(jnp.int32)) - 1
  above = s1[b_hi] - h1[b_hi]
  lo_hi = b_hi << SHIFT

  hist2, maxes = _sc_hist2(flat, jnp.full((L,), lo_hi, dtype=jnp.int32))
  s2 = _suffix_count(jnp.sum(hist2, axis=0))

  def resolve(rank):
    p = jnp.sum((s2 >= (rank - above)).astype(jnp.int32)) - 1
    return lax.bitcast_convert_type((lo_hi + p).astype(jnp.int32), jnp.float32)

  v_high = resolve(R_HIGH)
  v_low = jnp.where(s1[b_hi] >= R_LOW, resolve(R_LOW), jnp.max(maxes))
  t = v_low * 0.75 + v_high * 0.25

  return _compare(gray, t)


# speedup vs baseline: 18.9174x; 1.0798x over previous
import functools

import jax
import jax.numpy as jnp
from jax import lax
from jax.experimental import pallas as pl
from jax.experimental.pallas import tpu as pltpu
from jax.experimental.pallas import tpu_sc as plsc

B, C, H, W = 16, 3, 512, 512
N = B * H * W
NSUB = 32
PER_SUB = N // NSUB
CHUNK = 8192
NCHUNK = PER_SUB // CHUNK
L = 16
UNROLL = 8
SHIFT1 = 19
NB1 = 2048
H1_WORDS = NB1 * L
SHIFT2 = 8
NB2 = 2048
H2_WORDS = 2056 * L
NB3 = 256
H3_WORDS = 264 * L
R_HIGH = 10486
R_LOW = 10487

_mesh = plsc.VectorSubcoreMesh(
    core_axis_name="c", subcore_axis_name="s", num_cores=2, num_subcores=16
)
_sc_params = pltpu.CompilerParams(needs_layout_passes=False)


def _gray_body(img_ref, out_ref):
  r = img_ref[0, 0]
  g = img_ref[0, 1]
  b = img_ref[0, 2]
  out_ref[0, 0] = 0.2989 * r + 0.587 * g + 0.114 * b


def _grayscale(img):
  return pl.pallas_call(
      _gray_body,
      out_shape=jax.ShapeDtypeStruct((B, 1, H, W), jnp.float32),
      grid=(B,),
      in_specs=[pl.BlockSpec((1, C, H, W), lambda i: (i, 0, 0, 0))],
      out_specs=pl.BlockSpec((1, 1, H, W), lambda i: (i, 0, 0, 0)),
  )(img)


def _zero_hist(hist, nwords):
  zeros = jnp.zeros((L,), jnp.int32)

  def body(i, carry):
    base = i * L * UNROLL
    for u in range(UNROLL):
      hist[pl.ds(base + u * L, L)] = zeros
    return carry

  lax.fori_loop(0, nwords // (L * UNROLL), body, None)


def _stream_chunks(gray_hbm, base, buf0, buf1, sem0, sem1, process, carry):
  pltpu.async_copy(gray_hbm.at[pl.ds(base, CHUNK)], buf0, sem0)
  pltpu.async_copy(gray_hbm.at[pl.ds(base + CHUNK, CHUNK)], buf1, sem1)

  def wait(buf, sem):
    pltpu.make_async_copy(gray_hbm.at[pl.ds(0, CHUNK)], buf, sem).wait()

  def body(i, c2):
    c = 2 * i
    wait(buf0, sem0)
    c2 = process(buf0, c2)
    pltpu.async_copy(
        gray_hbm.at[pl.ds(base + (c + 2) * CHUNK, CHUNK)], buf0, sem0)
    wait(buf1, sem1)
    c2 = process(buf1, c2)
    pltpu.async_copy(
        gray_hbm.at[pl.ds(base + (c + 3) * CHUNK, CHUNK)], buf1, sem1)
    return c2

  carry = lax.fori_loop(0, NCHUNK // 2 - 1, body, carry)
  wait(buf0, sem0)
  carry = process(buf0, carry)
  wait(buf1, sem1)
  carry = process(buf1, carry)
  return carry


@functools.partial(
    pl.kernel,
    mesh=_mesh,
    out_type=jax.ShapeDtypeStruct((NSUB, H1_WORDS), jnp.int32),
    scratch_types=[
        pltpu.VMEM((CHUNK,), jnp.float32),
        pltpu.VMEM((CHUNK,), jnp.float32),
        pltpu.VMEM((H1_WORDS,), jnp.int32),
        pltpu.SemaphoreType.DMA,
        pltpu.SemaphoreType.DMA,
    ],
    compiler_params=_sc_params,
)
def _sc_round1(gray_hbm, out_hbm, buf0, buf1, hist, sem0, sem1):
  wid = lax.axis_index("s") * 2 + lax.axis_index("c")
  _zero_hist(hist, H1_WORDS)
  lane = lax.iota(jnp.int32, L)
  ones = jnp.ones((L,), jnp.int32)

  def process(buf, carry):
    def vec_body(v, c2):
      vbase = v * (L * UNROLL)
      for u in range(UNROLL):
        x = buf[pl.ds(vbase + u * L, L)]
        bits = plsc.bitcast(x, jnp.int32)
        idx = lax.shift_left(
            lax.shift_right_logical(bits, SHIFT1), 4) + lane
        plsc.addupdate_scatter(hist, [idx], ones)
      return c2

    return lax.fori_loop(0, CHUNK // (L * UNROLL), vec_body, carry)

  _stream_chunks(gray_hbm, wid * PER_SUB, buf0, buf1, sem0, sem1, process, 0)
  pltpu.sync_copy(hist, out_hbm.at[wid])


@functools.partial(
    pl.kernel,
    mesh=_mesh,
    out_type=jax.ShapeDtypeStruct((NSUB, H2_WORDS), jnp.int32),
    scratch_types=[
        pltpu.VMEM((CHUNK,), jnp.float32),
        pltpu.VMEM((CHUNK,), jnp.float32),
        pltpu.VMEM((L,), jnp.int32),
        pltpu.VMEM((H2_WORDS,), jnp.int32),
        pltpu.SemaphoreType.DMA,
        pltpu.SemaphoreType.DMA,
    ],
    compiler_params=_sc_params,
)
def _sc_round2(gray_hbm, lo_hbm, out_hbm, buf0, buf1, lov, hist, sem0, sem1):
  wid = lax.axis_index("s") * 2 + lax.axis_index("c")
  pltpu.sync_copy(lo_hbm, lov)
  _zero_hist(hist, H2_WORDS)
  lo = lov[...]
  lane16 = lax.iota(jnp.int32, L) + 16
  ones = jnp.ones((L,), jnp.int32)

  def process(buf, carry):
    def vec_body(v, c2):
      vbase = v * (L * UNROLL)
      for u in range(UNROLL):
        x = buf[pl.ds(vbase + u * L, L)]
        d = plsc.bitcast(x, jnp.int32) - lo
        i = lax.shift_right_arithmetic(d, SHIFT2)
        i = jnp.minimum(jnp.maximum(i, -1), NB2)
        idx = lax.shift_left(i, 4) + lane16
        plsc.addupdate_scatter(hist, [idx], ones)
      return c2

    return lax.fori_loop(0, CHUNK // (L * UNROLL), vec_body, carry)

  _stream_chunks(gray_hbm, wid * PER_SUB, buf0, buf1, sem0, sem1, process, 0)
  pltpu.sync_copy(hist, out_hbm.at[wid])


@functools.partial(
    pl.kernel,
    mesh=_mesh,
    out_type=(
        jax.ShapeDtypeStruct((NSUB, H3_WORDS), jnp.int32),
        jax.ShapeDtypeStruct((NSUB, L), jnp.float32),
    ),
    scratch_types=[
        pltpu.VMEM((CHUNK,), jnp.float32),
        pltpu.VMEM((CHUNK,), jnp.float32),
        pltpu.VMEM((L,), jnp.int32),
        pltpu.VMEM((H3_WORDS,), jnp.int32),
        pltpu.VMEM((L,), jnp.float32),
        pltpu.SemaphoreType.DMA,
        pltpu.SemaphoreType.DMA,
    ],
    compiler_params=_sc_params,
)
def _sc_round3(gray_hbm, lo_hbm, out_hbm, min_hbm,
               buf0, buf1, lov, hist, minv, sem0, sem1):
  wid = lax.axis_index("s") * 2 + lax.axis_index("c")
  pltpu.sync_copy(lo_hbm, lov)
  _zero_hist(hist, H3_WORDS)
  lo = lov[...]
  top = plsc.bitcast(lo + NB3, jnp.float32)
  lane16 = lax.iota(jnp.int32, L) + 16
  ones = jnp.ones((L,), jnp.int32)
  pos_inf = jnp.full((L,), jnp.inf, dtype=jnp.float32)

  def process(buf, acc):
    def vec_body(v, acc2):
      vbase = v * (L * UNROLL)
      for u in range(UNROLL):
        x = buf[pl.ds(vbase + u * L, L)]
        d = plsc.bitcast(x, jnp.int32) - lo
        i = jnp.minimum(jnp.maximum(d, -1), NB3)
        idx = lax.shift_left(i, 4) + lane16
        plsc.addupdate_scatter(hist, [idx], ones)
        acc2 = jnp.minimum(acc2, jnp.where(x >= top, x, pos_inf))
      return acc2

    return lax.fori_loop(0, CHUNK // (L * UNROLL), vec_body, acc)

  acc = _stream_chunks(
      gray_hbm, wid * PER_SUB, buf0, buf1, sem0, sem1, process, pos_inf)
  minv[...] = acc
  pltpu.sync_copy(hist, out_hbm.at[wid])
  pltpu.sync_copy(minv, min_hbm.at[wid])


def _cmp_body(t_ref, gray_ref, out_ref):
  out_ref[0, 0] = (gray_ref[0, 0] >= t_ref[0, 0]).astype(jnp.int32)


def _compare(gray, t):
  return pl.pallas_call(
      _cmp_body,
      out_shape=jax.ShapeDtypeStruct((B, 1, H, W), jnp.int32),
      grid=(B,),
      in_specs=[
          pl.BlockSpec(memory_space=pltpu.SMEM),
          pl.BlockSpec((1, 1, H, W), lambda i: (i, 0, 0, 0)),
      ],
      out_specs=pl.BlockSpec((1, 1, H, W), lambda i: (i, 0, 0, 0)),
  )(t.reshape(1, 1), gray)


def _suffix_count(hist):
  s = jnp.cumsum(hist[::-1])[::-1]
  return jnp.concatenate([s, jnp.zeros((1,), s.dtype)])


def _bcast(v):
  return jnp.full((L,), v, dtype=jnp.int32)


def kernel(img):
  gray = _grayscale(img)
  flat = gray.reshape(N)

  h1 = jnp.sum(_sc_round1(flat).reshape(NSUB, NB1, L), axis=(0, 2))
  s1 = _suffix_count(h1)
  b1 = jnp.sum((s1[:NB1] >= R_LOW).astype(jnp.int32)) - 1
  above1 = s1[b1 + 1]
  lo1 = b1 << SHIFT1

  h2 = jnp.sum(
      _sc_round2(flat, _bcast(lo1)).reshape(NSUB, 2056, L), axis=(0, 2)
  )[1:NB2 + 1]
  s2 = _suffix_count(h2)
  b2 = jnp.sum((s2[:NB2] >= (R_LOW - above1)).astype(jnp.int32)) - 1
  above2 = above1 + s2[b2 + 1]
  lo2 = lo1 + (b2 << SHIFT2)

  hist3, mins = _sc_round3(flat, _bcast(lo2))
  h3 = jnp.sum(hist3.reshape(NSUB, 264, L), axis=(0, 2))[1:NB3 + 1]
  s3 = _suffix_count(h3)

  p_lo = jnp.sum((s3[:NB3] >= (R_LOW - above2)).astype(jnp.int32)) - 1
  v_low = lax.bitcast_convert_type((lo2 + p_lo).astype(jnp.int32), jnp.float32)
  p_hi = jnp.sum((s3[:NB3] >= (R_HIGH - above2)).astype(jnp.int32)) - 1
  v_high = jnp.where(
      above2 >= R_HIGH,
      jnp.min(mins),
      lax.bitcast_convert_type((lo2 + p_hi).astype(jnp.int32), jnp.float32))
  t = v_low * 0.75 + v_high * 0.25

  return _compare(gray, t)
